# Initial kernel scaffold; baseline (speedup 1.0000x reference)
import jax
import jax.numpy as jnp
from jax.experimental import pallas as pl


def kernel(node_feat, coord_feat, edge_index, params):
    raise NotImplementedError("write your pallas kernel here")


# speedup vs baseline: 2.6184x; 2.6184x over previous
import functools

import jax
import jax.numpy as jnp
from jax import lax
from jax.experimental import pallas as pl
from jax.experimental.pallas import tpu as pltpu
from jax.experimental.pallas import tpu_sc as plsc

N_NODES = 10000
N_EDGES = 160000
D_FEAT = 128
H = 64
MSG_W = 80
CHUNK = 128
N_CHUNKS = N_EDGES // CHUNK
N_WORKERS = 32
ITERS = (N_CHUNKS + N_WORKERS - 1) // N_WORKERS
ROWS_PER_TILE = N_NODES // 16

_HIGH = jax.lax.Precision.HIGHEST


def _silu(v):
    return v * jax.nn.sigmoid(v)


def _dot(a, b):
    return jax.lax.dot_general(a, b, (((1,), (0,)), ((), ())),
                               preferred_element_type=jnp.float32)


def _sc_gather(hs, hd, xp, src, dst):
    mesh = plsc.VectorSubcoreMesh(core_axis_name="c", subcore_axis_name="s")

    @functools.partial(
        pl.kernel,
        out_type=(
            jax.ShapeDtypeStruct((N_EDGES, H), jnp.float32),
            jax.ShapeDtypeStruct((N_EDGES, H), jnp.float32),
            jax.ShapeDtypeStruct((N_EDGES, 16), jnp.float32),
            jax.ShapeDtypeStruct((N_EDGES, 16), jnp.float32),
        ),
        mesh=mesh,
        compiler_params=pltpu.CompilerParams(use_tc_tiling_on_sc=False),
        scratch_types=[
            pltpu.VMEM((CHUNK,), jnp.int32),
            pltpu.VMEM((CHUNK,), jnp.int32),
            pltpu.VMEM((CHUNK, H), jnp.float32),
            pltpu.VMEM((CHUNK, H), jnp.float32),
            pltpu.VMEM((CHUNK, 16), jnp.float32),
            pltpu.VMEM((CHUNK, 16), jnp.float32),
            pltpu.SemaphoreType.DMA,
        ],
    )
    def k(hs_hbm, hd_hbm, xp_hbm, src_hbm, dst_hbm,
          es_hbm, ed_hbm, xs_hbm, xd_hbm,
          isrc, idst, es_b, ed_b, xs_b, xd_b, sem):
        c = lax.axis_index("c")
        s = lax.axis_index("s")
        w = s * 2 + c

        def body(t, carry):
            j = w + jnp.int32(N_WORKERS) * t

            @pl.when(j < N_CHUNKS)
            def _():
                base = j * CHUNK
                pltpu.sync_copy(src_hbm.at[pl.ds(base, CHUNK)], isrc)
                pltpu.sync_copy(dst_hbm.at[pl.ds(base, CHUNK)], idst)
                d0 = pltpu.async_copy(hs_hbm.at[isrc], es_b, sem)
                d1 = pltpu.async_copy(hd_hbm.at[idst], ed_b, sem)
                d2 = pltpu.async_copy(xp_hbm.at[isrc], xs_b, sem)
                d3 = pltpu.async_copy(xp_hbm.at[idst], xd_b, sem)
                d0.wait()
                d1.wait()
                d2.wait()
                d3.wait()
                pltpu.sync_copy(es_b, es_hbm.at[pl.ds(base, CHUNK)])
                pltpu.sync_copy(ed_b, ed_hbm.at[pl.ds(base, CHUNK)])
                pltpu.sync_copy(xs_b, xs_hbm.at[pl.ds(base, CHUNK)])
                pltpu.sync_copy(xd_b, xd_hbm.at[pl.ds(base, CHUNK)])
            return carry

        lax.fori_loop(jnp.int32(0), jnp.int32(ITERS), body, jnp.int32(0))

    return k(hs, hd, xp, src, dst)


def _sc_scatter(msg, dst, zrows):
    mesh = plsc.VectorSubcoreMesh(core_axis_name="c", subcore_axis_name="s")

    @functools.partial(
        pl.kernel,
        out_type=jax.ShapeDtypeStruct((2, N_NODES, MSG_W), jnp.float32),
        mesh=mesh,
        compiler_params=pltpu.CompilerParams(use_tc_tiling_on_sc=False),
        scratch_types=[
            pltpu.VMEM_SHARED((N_NODES, MSG_W), jnp.float32),
            pltpu.VMEM((1, CHUNK), jnp.int32),
            pltpu.VMEM((CHUNK, MSG_W), jnp.float32),
        ],
    )
    def k(msg_hbm, dst_hbm, z_hbm, acc_hbm, acc_sh, idx2, msg_b):
        c = lax.axis_index("c")
        s = lax.axis_index("s")
        w = s * 2 + c

        r0 = s * ROWS_PER_TILE
        pltpu.sync_copy(z_hbm.at[pl.ds(r0, ROWS_PER_TILE)],
                        acc_sh.at[pl.ds(r0, ROWS_PER_TILE)])
        plsc.subcore_barrier()

        def body(t, carry):
            j = w + jnp.int32(N_WORKERS) * t

            @pl.when(j < N_CHUNKS)
            def _():
                base = j * CHUNK
                pltpu.sync_copy(dst_hbm.at[pl.ds(base, CHUNK)], idx2.at[jnp.int32(0)])
                pltpu.sync_copy(msg_hbm.at[pl.ds(base, CHUNK)], msg_b)
                pltpu.sync_copy(msg_b, acc_sh.at[idx2.at[jnp.int32(0)]], add=True)
            return carry

        lax.fori_loop(jnp.int32(0), jnp.int32(ITERS), body, jnp.int32(0))
        plsc.subcore_barrier()
        pltpu.sync_copy(acc_sh.at[pl.ds(r0, ROWS_PER_TILE)],
                        acc_hbm.at[c, pl.ds(r0, ROWS_PER_TILE)])

    return k(msg, dst, zrows)


_I0 = lambda: jnp.int32(0)


def _full(shape):
    return pl.BlockSpec(shape, lambda i: tuple(_I0() for _ in shape))


def _premix(h, wa, wb):
    blk = 2000
    grid = N_NODES // blk
    din = h.shape[1]

    def body(h_ref, wa_ref, wb_ref, hs_ref, hd_ref):
        hv = h_ref[...]
        hs_ref[...] = _dot(hv, wa_ref[...])
        hd_ref[...] = _dot(hv, wb_ref[...])

    return pl.pallas_call(
        body,
        grid=(grid,),
        in_specs=[
            pl.BlockSpec((blk, din), lambda i: (i, _I0())),
            _full((din, H)),
            _full((din, H)),
        ],
        out_specs=[
            pl.BlockSpec((blk, H), lambda i: (i, _I0())),
            pl.BlockSpec((blk, H), lambda i: (i, _I0())),
        ],
        out_shape=[
            jax.ShapeDtypeStruct((N_NODES, H), jnp.float32),
            jax.ShapeDtypeStruct((N_NODES, H), jnp.float32),
        ],
    )(h, wa, wb)


def _tc_edge(es, ed, xs, xd, w1r, b1, w2, b2, wc1, bc1, wc2):
    blk = 2000
    grid = N_EDGES // blk

    def body(es_ref, ed_ref, xs_ref, xd_ref, w1r_ref, b1_ref, w2_ref, b2_ref,
             wc1_ref, bc1_ref, wc2_ref, out_ref):
        diff = xs_ref[...] - xd_ref[...]
        radial = jnp.sum(diff * diff, axis=1, keepdims=True)
        inv = 1.0 / (jnp.sqrt(radial) + 1e-30)
        e1 = es_ref[...] + ed_ref[...] + radial * w1r_ref[...] + b1_ref[...]
        a1 = _silu(e1)
        mh = _silu(_dot(a1, w2_ref[...]) + b2_ref[...])
        c1 = _silu(_dot(mh, wc1_ref[...]) + bc1_ref[...])
        coef = _dot(c1, wc2_ref[...])
        mx = diff * (coef * inv)
        col = jax.lax.broadcasted_iota(jnp.int32, mx.shape, 1)
        mx = jnp.where(col == 3, 1.0, mx)
        out_ref[:, 0:H] = mh
        out_ref[:, H:MSG_W] = mx

    return pl.pallas_call(
        body,
        grid=(grid,),
        in_specs=[
            pl.BlockSpec((blk, H), lambda i: (i, _I0())),
            pl.BlockSpec((blk, H), lambda i: (i, _I0())),
            pl.BlockSpec((blk, 16), lambda i: (i, _I0())),
            pl.BlockSpec((blk, 16), lambda i: (i, _I0())),
            _full((1, H)),
            _full((1, H)),
            _full((H, H)),
            _full((1, H)),
            _full((H, H)),
            _full((1, H)),
            _full((H, 1)),
        ],
        out_specs=pl.BlockSpec((blk, MSG_W), lambda i: (i, _I0())),
        out_shape=jax.ShapeDtypeStruct((N_EDGES, MSG_W), jnp.float32),
    )(es, ed, xs, xd, w1r, b1, w2, b2, wc1, bc1, wc2)


def _tc_node(h, xp, acc, wn1a, wn1b, bn1, wn2, bn2, wna=None, wnb=None):
    blk = 2000
    grid = N_NODES // blk
    din = h.shape[1]
    with_next = wna is not None

    def body(*refs):
        if with_next:
            (h_ref, xp_ref, a0_ref, a1_ref, wn1a_ref, wn1b_ref, bn1_ref,
             wn2_ref, bn2_ref, wna_ref, wnb_ref,
             h_out, xp_out, hs_out, hd_out) = refs
        else:
            (h_ref, xp_ref, a0_ref, a1_ref, wn1a_ref, wn1b_ref, bn1_ref,
             wn2_ref, bn2_ref, h_out, xp_out) = refs
        a = a0_ref[0] + a1_ref[0]
        hn = a[:, 0:H]
        a16 = a[:, H:MSG_W]
        deg = a16[:, 3:4]
        scale = 1.0 / jnp.maximum(deg, 1.0)
        col = jax.lax.broadcasted_iota(jnp.int32, a16.shape, 1)
        xp_out[...] = jnp.where(col < 3, xp_ref[...] + a16 * scale, 0.0)
        t = _silu(_dot(h_ref[...], wn1a_ref[...]) + _dot(hn, wn1b_ref[...])
                  + bn1_ref[...])
        hh = _dot(t, wn2_ref[...]) + bn2_ref[...]
        h_out[...] = hh
        if with_next:
            hs_out[...] = _dot(hh, wna_ref[...])
            hd_out[...] = _dot(hh, wnb_ref[...])

    in_specs = [
        pl.BlockSpec((blk, din), lambda i: (i, _I0())),
        pl.BlockSpec((blk, 16), lambda i: (i, _I0())),
        pl.BlockSpec((1, blk, MSG_W), lambda i: (_I0(), i, _I0())),
        pl.BlockSpec((1, blk, MSG_W), lambda i: (_I0(), i, _I0())),
        _full((din, H)),
        _full((H, H)),
        _full((1, H)),
        _full((H, H)),
        _full((1, H)),
    ]
    out_specs = [
        pl.BlockSpec((blk, H), lambda i: (i, _I0())),
        pl.BlockSpec((blk, 16), lambda i: (i, _I0())),
    ]
    out_shape = [
        jax.ShapeDtypeStruct((N_NODES, H), jnp.float32),
        jax.ShapeDtypeStruct((N_NODES, 16), jnp.float32),
    ]
    args = [h, xp, acc[0:1], acc[1:2], wn1a, wn1b, bn1, wn2, bn2]
    if with_next:
        in_specs += [_full((H, H)), _full((H, H))]
        out_specs += [pl.BlockSpec((blk, H), lambda i: (i, _I0())),
                      pl.BlockSpec((blk, H), lambda i: (i, _I0()))]
        out_shape += [jax.ShapeDtypeStruct((N_NODES, H), jnp.float32),
                      jax.ShapeDtypeStruct((N_NODES, H), jnp.float32)]
        args += [wna, wnb]

    return pl.pallas_call(
        body,
        grid=(grid,),
        in_specs=in_specs,
        out_specs=out_specs,
        out_shape=out_shape,
    )(*args)


def _tc_final(h, nm1w, nm1b, nm2w, nm2b, pm1w, pm1b, pm2w, pm2b):
    blk = 2000
    grid = N_NODES // blk

    def body(h_ref, nm1w_ref, nm1b_ref, nm2w_ref, nm2b_ref,
             pm1w_ref, pm1b_ref, pm2w_ref, pm2b_ref, out_ref, acc_ref):
        i = pl.program_id(0)

        @pl.when(i == 0)
        def _():
            acc_ref[...] = jnp.zeros_like(acc_ref)

        sblk = _silu(_dot(h_ref[...], nm1w_ref[...]) + nm1b_ref[...])
        acc_ref[...] += jnp.sum(sblk, axis=0, keepdims=True)

        @pl.when(i == grid - 1)
        def _():
            g = _dot(acc_ref[...], nm2w_ref[...]) + N_NODES * nm2b_ref[...]
            p = _silu(_dot(g, pm1w_ref[...]) + pm1b_ref[...])
            out_ref[...] = _dot(p, pm2w_ref[...]) + pm2b_ref[...]

    return pl.pallas_call(
        body,
        grid=(grid,),
        in_specs=[
            pl.BlockSpec((blk, H), lambda i: (i, _I0())),
            _full((H, H)),
            _full((1, H)),
            _full((H, H)),
            _full((1, H)),
            _full((H, H)),
            _full((1, H)),
            _full((H, 1)),
            _full((1, 1)),
        ],
        out_specs=_full((1, 1)),
        out_shape=jax.ShapeDtypeStruct((1, 1), jnp.float32),
        scratch_shapes=[pltpu.VMEM((1, H), jnp.float32)],
    )(h, nm1w, nm1b, nm2w, nm2b, pm1w, pm1b, pm2w, pm2b)


def kernel(node_feat, coord_feat, edge_index, params):
    f32 = jnp.float32
    node_feat = node_feat.astype(f32)
    src = edge_index[0].astype(jnp.int32)
    dst = edge_index[1].astype(jnp.int32)
    xp = jnp.zeros((N_NODES, 16), f32).at[:, 0:3].set(coord_feat.astype(f32))
    zrows = jnp.zeros((N_NODES, MSG_W), f32)

    layers = params["layers"]
    w1_0 = layers[0]["edge1"]["W"].astype(f32)
    hs, hd = _premix(node_feat, w1_0[0:D_FEAT], w1_0[D_FEAT:2 * D_FEAT])

    h = node_feat
    for i, lp in enumerate(layers):
        din = D_FEAT if i == 0 else H
        w1 = lp["edge1"]["W"].astype(f32)
        w1r = w1[2 * din:2 * din + 1]
        b1 = lp["edge1"]["b"].astype(f32)[None]
        es, ed, xs, xd = _sc_gather(hs, hd, xp, src, dst)
        msg = _tc_edge(es, ed, xs, xd, w1r, b1,
                       lp["edge2"]["W"].astype(f32),
                       lp["edge2"]["b"].astype(f32)[None],
                       lp["coord1"]["W"].astype(f32),
                       lp["coord1"]["b"].astype(f32)[None],
                       lp["coord2"]["W"].astype(f32))
        acc = _sc_scatter(msg, dst, zrows)
        wn1 = lp["node1"]["W"].astype(f32)
        if i + 1 < len(layers):
            wnxt = layers[i + 1]["edge1"]["W"].astype(f32)
            h, xp, hs, hd = _tc_node(
                h, xp, acc, wn1[0:din], wn1[din:din + H],
                lp["node1"]["b"].astype(f32)[None],
                lp["node2"]["W"].astype(f32),
                lp["node2"]["b"].astype(f32)[None],
                wnxt[0:H], wnxt[H:2 * H])
        else:
            h, xp = _tc_node(
                h, xp, acc, wn1[0:din], wn1[din:din + H],
                lp["node1"]["b"].astype(f32)[None],
                lp["node2"]["W"].astype(f32),
                lp["node2"]["b"].astype(f32)[None])

    return _tc_final(
        h,
        params["nm1"]["W"].astype(f32), params["nm1"]["b"].astype(f32)[None],
        params["nm2"]["W"].astype(f32), params["nm2"]["b"].astype(f32)[None],
        params["pm1"]["W"].astype(f32), params["pm1"]["b"].astype(f32)[None],
        params["pm2"]["W"].astype(f32), params["pm2"]["b"].astype(f32)[None])
